# Initial kernel scaffold; baseline (speedup 1.0000x reference)
import jax
import jax.numpy as jnp
from jax.experimental import pallas as pl


def kernel(x, pe_table):
    raise NotImplementedError("write your pallas kernel here")


# speedup vs baseline: 1.6697x; 1.6697x over previous
import jax
import jax.numpy as jnp
from jax.experimental import pallas as pl


def _add_block(x_ref, pe_ref, o_ref):
    o_ref[...] = x_ref[...] + pe_ref[...]


def kernel(x, pe_table):
    batch, seq_len, d_model = x.shape
    bs = 512
    while seq_len % bs != 0:
        bs //= 2
    grid = (seq_len // bs, batch)
    return pl.pallas_call(
        _add_block,
        grid=grid,
        in_specs=[
            pl.BlockSpec((1, bs, d_model), lambda s, b: (b, s, 0)),
            pl.BlockSpec((bs, d_model), lambda s, b: (s, 0)),
        ],
        out_specs=pl.BlockSpec((1, bs, d_model), lambda s, b: (b, s, 0)),
        out_shape=jax.ShapeDtypeStruct(x.shape, x.dtype),
    )(x, pe_table)
